# baseline (speedup 1.0000x reference)
import jax
import jax.numpy as jnp
from jax.experimental import pallas as pl

_NUM_ENTRY = 1000
_E_PAD = 1024
_D = 256
_LOSS_SCALE = 0.1 + 0.2


def _vq_block(f_ref, e_ref, q_ref, idx_ref, loss_ref):
    f = f_ref[...]
    e = e_ref[...]
    scores = jax.lax.dot_general(
        f, e, (((1,), (1,)), ((), ())),
        preferred_element_type=jnp.float32,
        precision=jax.lax.Precision.DEFAULT)
    en = jnp.sum(e * e, axis=1)[None, :]
    fn = jnp.sum(f * f, axis=1, keepdims=True)
    dist = (fn + en) - 2.0 * scores
    col = jax.lax.broadcasted_iota(jnp.int32, dist.shape, 1)
    dist = jnp.where(col < _NUM_ENTRY, dist, jnp.inf)
    m = jnp.min(dist, axis=1, keepdims=True)
    idx = jnp.min(jnp.where(dist == m, col, _E_PAD), axis=1, keepdims=True)
    loss_ref[...] = m * (_LOSS_SCALE / _D)
    idx_ref[...] = idx
    one_hot = (col == idx).astype(jnp.float32)
    q = jax.lax.dot_general(
        one_hot, e, (((1,), (0,)), ((), ())),
        preferred_element_type=jnp.float32,
        precision=jax.lax.Precision.HIGHEST)
    q_ref[...] = f + (q - f)


def kernel(h, emb_weight):
    M = h.shape[0] * h.shape[1]
    flat = h.reshape(M, _D)
    e = jnp.pad(emb_weight, ((0, _E_PAD - _NUM_ENTRY), (0, 0)))
    BM = 1024
    q, idx, loss = pl.pallas_call(
        _vq_block,
        grid=(M // BM,),
        in_specs=[
            pl.BlockSpec((BM, _D), lambda i: (i, 0)),
            pl.BlockSpec((_E_PAD, _D), lambda i: (0, 0)),
        ],
        out_specs=[
            pl.BlockSpec((BM, _D), lambda i: (i, 0)),
            pl.BlockSpec((BM, 1), lambda i: (i, 0)),
            pl.BlockSpec((BM, 1), lambda i: (i, 0)),
        ],
        out_shape=[
            jax.ShapeDtypeStruct((M, _D), jnp.float32),
            jax.ShapeDtypeStruct((M, 1), jnp.int32),
            jax.ShapeDtypeStruct((M, 1), jnp.float32),
        ],
    )(flat, e)
    return (q.reshape(h.shape), idx, loss[:, 0])


# speedup vs baseline: 1.7818x; 1.7818x over previous
import jax
import jax.numpy as jnp
from jax.experimental import pallas as pl

_NUM_ENTRY = 1000
_E_PAD = 1024
_D = 256
_LOSS_SCALE = 0.1 + 0.2


def _vq_block(f_ref, e_ref, q_ref, idx_ref, loss_ref):
    f = f_ref[...]
    e = e_ref[...]
    scores = jax.lax.dot_general(
        f, e, (((1,), (1,)), ((), ())),
        preferred_element_type=jnp.float32,
        precision=jax.lax.Precision.DEFAULT)
    en = jnp.sum(e * e, axis=1)[None, :]
    fn = jnp.sum(f * f, axis=1, keepdims=True)
    dist = (fn + en) - 2.0 * scores
    col = jax.lax.broadcasted_iota(jnp.int32, dist.shape, 1)
    dist = jnp.where(col < _NUM_ENTRY, dist, jnp.inf)
    m = jnp.min(dist, axis=1, keepdims=True)
    idx = jnp.min(jnp.where(dist == m, col, _E_PAD), axis=1, keepdims=True)
    loss_ref[...] = m * (_LOSS_SCALE / _D)
    idx_ref[...] = idx
    one_hot = (col == idx).astype(jnp.float32)
    q = jax.lax.dot_general(
        one_hot, e, (((1,), (0,)), ((), ())),
        preferred_element_type=jnp.float32,
        precision=jax.lax.Precision.DEFAULT)
    q_ref[...] = f + (q - f)


def kernel(h, emb_weight):
    M = h.shape[0] * h.shape[1]
    flat = h.reshape(M, _D)
    e = jnp.pad(emb_weight, ((0, _E_PAD - _NUM_ENTRY), (0, 0)))
    BM = 1024
    q, idx, loss = pl.pallas_call(
        _vq_block,
        grid=(M // BM,),
        in_specs=[
            pl.BlockSpec((BM, _D), lambda i: (i, 0)),
            pl.BlockSpec((_E_PAD, _D), lambda i: (0, 0)),
        ],
        out_specs=[
            pl.BlockSpec((BM, _D), lambda i: (i, 0)),
            pl.BlockSpec((BM, 1), lambda i: (i, 0)),
            pl.BlockSpec((BM, 1), lambda i: (i, 0)),
        ],
        out_shape=[
            jax.ShapeDtypeStruct((M, _D), jnp.float32),
            jax.ShapeDtypeStruct((M, 1), jnp.int32),
            jax.ShapeDtypeStruct((M, 1), jnp.float32),
        ],
    )(flat, e)
    return (q.reshape(h.shape), idx, loss[:, 0])


# speedup vs baseline: 1.8878x; 1.0595x over previous
import functools

import jax
import jax.numpy as jnp
from jax import lax
from jax.experimental import pallas as pl
from jax.experimental.pallas import tpu as pltpu
from jax.experimental.pallas import tpu_sc as plsc

_NUM_ENTRY = 1000
_E_PAD = 1024
_D = 256
_LOSS_SCALE = 0.1 + 0.2

_NC = 2
_NS = 16
_NW = _NC * _NS
_CHUNK = 128


def _vq_dist_block(f_ref, e_ref, idx_ref, loss_ref):
    f = f_ref[...]
    e = e_ref[...]
    scores = jax.lax.dot_general(
        f, e, (((1,), (1,)), ((), ())),
        preferred_element_type=jnp.float32,
        precision=jax.lax.Precision.DEFAULT)
    row = lax.broadcasted_iota(jnp.int32, (1, _E_PAD), 1)
    en = jnp.sum(e * e, axis=1)[None, :]
    en = jnp.where(row < _NUM_ENTRY, en, jnp.inf)
    fn = jnp.sum(f * f, axis=1, keepdims=True)
    dist = (fn + en) - 2.0 * scores
    m = jnp.min(dist, axis=1, keepdims=True)
    col = lax.broadcasted_iota(jnp.int32, dist.shape, 1)
    idx = jnp.min(jnp.where(dist == m, col, _E_PAD), axis=1, keepdims=True)
    loss_ref[...] = m * (_LOSS_SCALE / _D)
    idx_ref[...] = idx


def _sc_gather_body(emb_hbm, idx_hbm, out_hbm, idx_v, rows_v, sem):
    wid = lax.axis_index("s") * _NC + lax.axis_index("c")
    n_rows = out_hbm.shape[0]
    b_per_w = n_rows // _NW
    base = wid * b_per_w
    for c in range(b_per_w // _CHUNK):
        row0 = base + c * _CHUNK
        pltpu.sync_copy(idx_hbm.at[pl.ds(row0, _CHUNK)], idx_v)
        pltpu.async_copy(emb_hbm.at[idx_v], rows_v, sem).wait()
        pltpu.sync_copy(rows_v, out_hbm.at[pl.ds(row0, _CHUNK)])


def kernel(h, emb_weight):
    M = h.shape[0] * h.shape[1]
    flat = h.reshape(M, _D)
    e = jnp.pad(emb_weight, ((0, _E_PAD - _NUM_ENTRY), (0, 0)))
    BM = 1024
    idx, loss = pl.pallas_call(
        _vq_dist_block,
        grid=(M // BM,),
        in_specs=[
            pl.BlockSpec((BM, _D), lambda i: (i, 0)),
            pl.BlockSpec((_E_PAD, _D), lambda i: (0, 0)),
        ],
        out_specs=[
            pl.BlockSpec((BM, 1), lambda i: (i, 0)),
            pl.BlockSpec((BM, 1), lambda i: (i, 0)),
        ],
        out_shape=[
            jax.ShapeDtypeStruct((M, 1), jnp.int32),
            jax.ShapeDtypeStruct((M, 1), jnp.float32),
        ],
    )(flat, e)

    gather = functools.partial(
        pl.kernel,
        out_type=jax.ShapeDtypeStruct((M, _D), jnp.float32),
        mesh=plsc.VectorSubcoreMesh(core_axis_name="c", subcore_axis_name="s"),
        scratch_types=[
            pltpu.VMEM((_CHUNK,), jnp.int32),
            pltpu.VMEM((_CHUNK, _D), jnp.float32),
            pltpu.SemaphoreType.DMA,
        ],
    )(_sc_gather_body)
    q = gather(emb_weight, idx.reshape(M))
    return (q.reshape(h.shape), idx, loss[:, 0])


# speedup vs baseline: 1.9141x; 1.0139x over previous
import functools

import jax
import jax.numpy as jnp
from jax import lax
from jax.experimental import pallas as pl
from jax.experimental.pallas import tpu as pltpu
from jax.experimental.pallas import tpu_sc as plsc

_NUM_ENTRY = 1000
_E_PAD = 1024
_D = 256
_LOSS_SCALE = 0.1 + 0.2

_NC = 2
_NS = 16
_NW = _NC * _NS
_CHUNK = 128


def _vq_dist_block(f_ref, e_ref, idx_ref, loss_ref):
    f = f_ref[...]
    e = e_ref[...]
    scores2 = jax.lax.dot_general(
        f, e + e, (((1,), (1,)), ((), ())),
        preferred_element_type=jnp.float32,
        precision=jax.lax.Precision.DEFAULT)
    row = lax.broadcasted_iota(jnp.int32, (1, _E_PAD), 1)
    en = jnp.sum(e * e, axis=1)[None, :]
    en = jnp.where(row < _NUM_ENTRY, en, jnp.inf)
    fn = jnp.sum(f * f, axis=1, keepdims=True)
    dist = (fn + en) - scores2
    m = jnp.min(dist, axis=1, keepdims=True)
    col = lax.broadcasted_iota(jnp.int32, dist.shape, 1)
    idx = jnp.min(jnp.where(dist == m, col, _E_PAD), axis=1, keepdims=True)
    loss_ref[...] = m * (_LOSS_SCALE / _D)
    idx_ref[...] = idx


def _sc_gather_body(emb_hbm, idx_hbm, out_hbm,
                    idx_v, rows_a, rows_b, ga, gb, sa, sb):
    wid = lax.axis_index("s") * _NC + lax.axis_index("c")
    n_rows = out_hbm.shape[0]
    b_per_w = n_rows // _NW
    base = wid * b_per_w
    n_chunks = b_per_w // _CHUNK
    pltpu.sync_copy(idx_hbm.at[pl.ds(base, b_per_w)], idx_v)
    rows = (rows_a, rows_b)
    gsem = (ga, gb)
    ssem = (sa, sb)

    def gather_start(c):
        b = c % 2
        return pltpu.async_copy(
            emb_hbm.at[idx_v.at[pl.ds(c * _CHUNK, _CHUNK)]], rows[b], gsem[b])

    gath = [gather_start(0), None]
    scat = [None, None]
    for c in range(n_chunks):
        b = c % 2
        nb = (c + 1) % 2
        if c + 1 < n_chunks:
            if scat[nb] is not None:
                scat[nb].wait()
            gath[nb] = gather_start(c + 1)
        gath[b].wait()
        scat[b] = pltpu.async_copy(
            rows[b], out_hbm.at[pl.ds(base + c * _CHUNK, _CHUNK)], ssem[b])
    scat[0].wait()
    scat[1].wait()


def kernel(h, emb_weight):
    M = h.shape[0] * h.shape[1]
    flat = h.reshape(M, _D)
    e = jnp.pad(emb_weight, ((0, _E_PAD - _NUM_ENTRY), (0, 0)))
    BM = 1024
    idx, loss = pl.pallas_call(
        _vq_dist_block,
        grid=(M // BM,),
        in_specs=[
            pl.BlockSpec((BM, _D), lambda i: (i, 0)),
            pl.BlockSpec((_E_PAD, _D), lambda i: (0, 0)),
        ],
        out_specs=[
            pl.BlockSpec((BM, 1), lambda i: (i, 0)),
            pl.BlockSpec((BM, 1), lambda i: (i, 0)),
        ],
        out_shape=[
            jax.ShapeDtypeStruct((M, 1), jnp.int32),
            jax.ShapeDtypeStruct((M, 1), jnp.float32),
        ],
    )(flat, e)

    gather = functools.partial(
        pl.kernel,
        out_type=jax.ShapeDtypeStruct((M, _D), jnp.float32),
        mesh=plsc.VectorSubcoreMesh(core_axis_name="c", subcore_axis_name="s"),
        scratch_types=[
            pltpu.VMEM((M // _NW,), jnp.int32),
            pltpu.VMEM((_CHUNK, _D), jnp.float32),
            pltpu.VMEM((_CHUNK, _D), jnp.float32),
            pltpu.SemaphoreType.DMA,
            pltpu.SemaphoreType.DMA,
            pltpu.SemaphoreType.DMA,
            pltpu.SemaphoreType.DMA,
        ],
    )(_sc_gather_body)
    q = gather(emb_weight, idx.reshape(M))
    return (q.reshape(h.shape), idx, loss[:, 0])


# speedup vs baseline: 2.0937x; 1.0938x over previous
import functools

import jax
import jax.numpy as jnp
from jax import lax
from jax.experimental import pallas as pl
from jax.experimental.pallas import tpu as pltpu
from jax.experimental.pallas import tpu_sc as plsc

_NUM_ENTRY = 1000
_E_PAD = 1024
_D = 256
_LOSS_SCALE = 0.1 + 0.2

_NC = 2
_NS = 16
_NW = _NC * _NS
_CHUNK = 128


def _vq_dist_block(f_ref, e_ref, idx_ref, loss_ref):
    f = f_ref[...]
    e = e_ref[...]
    scores2 = jax.lax.dot_general(
        f, e + e, (((1,), (1,)), ((), ())),
        preferred_element_type=jnp.float32,
        precision=jax.lax.Precision.DEFAULT)
    row = lax.broadcasted_iota(jnp.int32, (1, _E_PAD), 1)
    en = jnp.sum(e * e, axis=1)[None, :]
    en = jnp.where(row < _NUM_ENTRY, en, jnp.inf)
    fn = jnp.sum(f * f, axis=1, keepdims=True)
    dist = (fn + en) - scores2
    m = jnp.min(dist, axis=1, keepdims=True)
    col = lax.broadcasted_iota(jnp.int32, dist.shape, 1)
    idx = jnp.min(jnp.where(dist == m, col, _E_PAD), axis=1, keepdims=True)
    loss_ref[...] = m * (_LOSS_SCALE / _D)
    idx_ref[...] = idx


def _sc_gather_body(emb_hbm, idx_hbm, out_hbm,
                    idx_v, rows_a, rows_b, ga, gb, sa, sb):
    wid = lax.axis_index("s") * _NC + lax.axis_index("c")
    n_rows = out_hbm.shape[0]
    b_per_w = n_rows // _NW
    base = wid * b_per_w
    n_chunks = b_per_w // _CHUNK
    pltpu.sync_copy(idx_hbm.at[pl.ds(base, b_per_w)], idx_v)
    rows = (rows_a, rows_b)
    gsem = (ga, gb)
    ssem = (sa, sb)

    def gather_start(c):
        b = c % 2
        return pltpu.async_copy(
            emb_hbm.at[idx_v.at[pl.ds(c * _CHUNK, _CHUNK)]], rows[b], gsem[b])

    gath = [gather_start(0), None]
    scat = [None, None]
    for c in range(n_chunks):
        b = c % 2
        nb = (c + 1) % 2
        if c + 1 < n_chunks:
            if scat[nb] is not None:
                scat[nb].wait()
            gath[nb] = gather_start(c + 1)
        gath[b].wait()
        scat[b] = pltpu.async_copy(
            rows[b], out_hbm.at[pl.ds(base + c * _CHUNK, _CHUNK)], ssem[b])
    scat[0].wait()
    scat[1].wait()


def kernel(h, emb_weight):
    M = h.shape[0] * h.shape[1]
    flat = h.reshape(M, _D)
    e = jnp.pad(emb_weight, ((0, _E_PAD - _NUM_ENTRY), (0, 0)))
    BM = 4096
    idx, loss = pl.pallas_call(
        _vq_dist_block,
        grid=(M // BM,),
        in_specs=[
            pl.BlockSpec((BM, _D), lambda i: (i, 0)),
            pl.BlockSpec((_E_PAD, _D), lambda i: (0, 0)),
        ],
        out_specs=[
            pl.BlockSpec((BM, 1), lambda i: (i, 0)),
            pl.BlockSpec((BM, 1), lambda i: (i, 0)),
        ],
        out_shape=[
            jax.ShapeDtypeStruct((M, 1), jnp.int32),
            jax.ShapeDtypeStruct((M, 1), jnp.float32),
        ],
    )(flat, e)

    gather = functools.partial(
        pl.kernel,
        out_type=jax.ShapeDtypeStruct((M, _D), jnp.float32),
        mesh=plsc.VectorSubcoreMesh(core_axis_name="c", subcore_axis_name="s"),
        scratch_types=[
            pltpu.VMEM((M // _NW,), jnp.int32),
            pltpu.VMEM((_CHUNK, _D), jnp.float32),
            pltpu.VMEM((_CHUNK, _D), jnp.float32),
            pltpu.SemaphoreType.DMA,
            pltpu.SemaphoreType.DMA,
            pltpu.SemaphoreType.DMA,
            pltpu.SemaphoreType.DMA,
        ],
    )(_sc_gather_body)
    q = gather(emb_weight, idx.reshape(M))
    return (q.reshape(h.shape), idx, loss[:, 0])
